# Initial kernel scaffold; baseline (speedup 1.0000x reference)
import jax
import jax.numpy as jnp
from jax.experimental import pallas as pl


def kernel(input, shuffle_indices):
    raise NotImplementedError("write your pallas kernel here")


# speedup vs baseline: 1.3113x; 1.3113x over previous
import functools

import jax
import jax.numpy as jnp
from jax import lax
from jax.experimental import pallas as pl
from jax.experimental.pallas import tpu as pltpu
from jax.experimental.pallas import tpu_sc as plsc

N_ROWS = 131072
N_COLS = 128

_info = plsc.get_sparse_core_info()
NC, NS, L = _info.num_cores, _info.num_subcores, _info.num_lanes
NW = NC * NS
ROWS_PER_W = N_ROWS // NW
CHUNK_ROWS = 256
N_CHUNKS = ROWS_PER_W // CHUNK_ROWS
CHUNK_ELEMS = CHUNK_ROWS * N_COLS
G = N_COLS // L

_mesh = plsc.VectorSubcoreMesh(core_axis_name="c", subcore_axis_name="s")


@functools.partial(
    pl.kernel,
    mesh=_mesh,
    out_type=jax.ShapeDtypeStruct((N_ROWS * N_COLS,), jnp.float32),
    scratch_types=[
        pltpu.VMEM((N_COLS,), jnp.int32),
        pltpu.VMEM((CHUNK_ELEMS,), jnp.float32),
        pltpu.VMEM((CHUNK_ELEMS,), jnp.float32),
    ],
    compiler_params=pltpu.CompilerParams(needs_layout_passes=False),
)
def _shuffle(x_hbm, idx_hbm, out_hbm, idx_v, in_v, out_v):
    wid = lax.axis_index("s") * NC + lax.axis_index("c")
    pltpu.sync_copy(idx_hbm, idx_v)
    col_idx = [idx_v[pl.ds(g * L, L)] for g in range(G)]
    base_w = wid * (ROWS_PER_W * N_COLS)

    def chunk_body(ci, carry):
        base = base_w + ci * CHUNK_ELEMS
        pltpu.sync_copy(x_hbm.at[pl.ds(base, CHUNK_ELEMS)], in_v)

        def row_body(r, c2):
            rb = r * N_COLS
            for g in range(G):
                v = plsc.load_gather(in_v, [col_idx[g] + rb])
                out_v[pl.ds(rb + g * L, L)] = v
            return c2

        lax.fori_loop(0, CHUNK_ROWS, row_body, 0)
        pltpu.sync_copy(out_v, out_hbm.at[pl.ds(base, CHUNK_ELEMS)])
        return carry

    lax.fori_loop(0, N_CHUNKS, chunk_body, 0)


def kernel(input, shuffle_indices):
    out_flat = _shuffle(input.reshape(-1), shuffle_indices)
    return out_flat.reshape(N_ROWS, N_COLS)


# speedup vs baseline: 1.6936x; 1.2915x over previous
import functools

import jax
import jax.numpy as jnp
from jax import lax
from jax.experimental import pallas as pl
from jax.experimental.pallas import tpu as pltpu
from jax.experimental.pallas import tpu_sc as plsc

N_ROWS = 131072
N_COLS = 128

_info = plsc.get_sparse_core_info()
NC, NS, L = _info.num_cores, _info.num_subcores, _info.num_lanes
NW = NC * NS
ROWS_PER_W = N_ROWS // NW
CHUNK_ROWS = 128
N_CHUNKS = ROWS_PER_W // CHUNK_ROWS
CHUNK_ELEMS = CHUNK_ROWS * N_COLS
G = N_COLS // L
NBUF = 2

_mesh = plsc.VectorSubcoreMesh(core_axis_name="c", subcore_axis_name="s")


@functools.partial(
    pl.kernel,
    mesh=_mesh,
    out_type=jax.ShapeDtypeStruct((N_ROWS * N_COLS,), jnp.float32),
    scratch_types=[
        pltpu.VMEM((N_COLS,), jnp.int32),
        [pltpu.VMEM((CHUNK_ELEMS,), jnp.float32) for _ in range(NBUF)],
        [pltpu.VMEM((CHUNK_ELEMS,), jnp.float32) for _ in range(NBUF)],
        [pltpu.SemaphoreType.DMA for _ in range(NBUF)],
        [pltpu.SemaphoreType.DMA for _ in range(NBUF)],
    ],
    compiler_params=pltpu.CompilerParams(needs_layout_passes=False),
)
def _shuffle(x_hbm, idx_hbm, out_hbm, idx_v, in_v, out_v, in_sem, out_sem):
    wid = lax.axis_index("s") * NC + lax.axis_index("c")
    pltpu.sync_copy(idx_hbm, idx_v)
    col_idx = [idx_v[pl.ds(g * L, L)] for g in range(G)]
    base_w = wid * (ROWS_PER_W * N_COLS)

    def in_copy(ci, b):
        return pltpu.async_copy(
            x_hbm.at[pl.ds(base_w + ci * CHUNK_ELEMS, CHUNK_ELEMS)],
            in_v[b], in_sem[b])

    def permute_chunk(src, dst):
        def row_body(r, c2):
            rb = r * N_COLS
            for g in range(G):
                v = plsc.load_gather(src, [col_idx[g] + rb])
                dst[pl.ds(rb + g * L, L)] = v
            return c2

        lax.fori_loop(0, CHUNK_ROWS, row_body, 0)

    in_d = [None] * NBUF
    out_d = [None] * NBUF
    in_d[0] = in_copy(0, 0)
    for ci in range(N_CHUNKS):
        b = ci % NBUF
        in_d[b].wait()
        if ci + 1 < N_CHUNKS:
            nb = (ci + 1) % NBUF
            in_d[nb] = in_copy(ci + 1, nb)
        if out_d[b] is not None:
            out_d[b].wait()
        permute_chunk(in_v[b], out_v[b])
        out_d[b] = pltpu.async_copy(
            out_v[b],
            out_hbm.at[pl.ds(base_w + ci * CHUNK_ELEMS, CHUNK_ELEMS)],
            out_sem[b])
    for b in range(NBUF):
        if out_d[b] is not None:
            out_d[b].wait()


def kernel(input, shuffle_indices):
    out_flat = _shuffle(input.reshape(-1), shuffle_indices)
    return out_flat.reshape(N_ROWS, N_COLS)


# speedup vs baseline: 3.2208x; 1.9018x over previous
import functools

import jax
import jax.numpy as jnp
from jax import lax
from jax.experimental import pallas as pl
from jax.experimental.pallas import tpu as pltpu
from jax.experimental.pallas import tpu_sc as plsc

N_ROWS = 131072
N_COLS = 128

_info = plsc.get_sparse_core_info()
NC, NS, L = _info.num_cores, _info.num_subcores, _info.num_lanes
NW = NC * NS
ROWS_PER_W = N_ROWS // NW
CHUNK_ROWS = 128
N_CHUNKS = ROWS_PER_W // CHUNK_ROWS
CHUNK_ELEMS = CHUNK_ROWS * N_COLS
G = N_COLS // L
NBUF = 2

_mesh = plsc.VectorSubcoreMesh(core_axis_name="c", subcore_axis_name="s")


@functools.partial(
    pl.kernel,
    mesh=_mesh,
    out_type=jax.ShapeDtypeStruct((N_ROWS * N_COLS,), jnp.float32),
    scratch_types=[
        pltpu.VMEM((N_COLS,), jnp.int32),
        [pltpu.VMEM((CHUNK_ELEMS,), jnp.float32) for _ in range(NBUF)],
        [pltpu.VMEM((CHUNK_ELEMS,), jnp.float32) for _ in range(NBUF)],
        [pltpu.SemaphoreType.DMA for _ in range(NBUF)],
        [pltpu.SemaphoreType.DMA for _ in range(NBUF)],
    ],
    compiler_params=pltpu.CompilerParams(needs_layout_passes=False),
)
def _shuffle(x_hbm, idx_hbm, out_hbm, idx_v, in_v, out_v, in_sem, out_sem):
    wid = lax.axis_index("s") * NC + lax.axis_index("c")
    pltpu.sync_copy(idx_hbm, idx_v)
    col_idx = [idx_v[pl.ds(g * L, L)] for g in range(G)]
    base_w = wid * (ROWS_PER_W * N_COLS)

    def in_copy(ci, b):
        return pltpu.async_copy(
            x_hbm.at[pl.ds(base_w + ci * CHUNK_ELEMS, CHUNK_ELEMS)],
            in_v[b], in_sem[b])

    def permute_chunk(src, dst):
        @plsc.parallel_loop(0, CHUNK_ROWS, step=1, unroll=4)
        def row_body(r):
            rb = r * N_COLS
            for g in range(G):
                v = plsc.load_gather(src, [col_idx[g] + rb])
                dst[pl.ds(rb + g * L, L)] = v

    in_d = [None] * NBUF
    out_d = [None] * NBUF
    in_d[0] = in_copy(0, 0)
    for ci in range(N_CHUNKS):
        b = ci % NBUF
        in_d[b].wait()
        if ci + 1 < N_CHUNKS:
            nb = (ci + 1) % NBUF
            in_d[nb] = in_copy(ci + 1, nb)
        if out_d[b] is not None:
            out_d[b].wait()
        permute_chunk(in_v[b], out_v[b])
        out_d[b] = pltpu.async_copy(
            out_v[b],
            out_hbm.at[pl.ds(base_w + ci * CHUNK_ELEMS, CHUNK_ELEMS)],
            out_sem[b])
    for b in range(NBUF):
        if out_d[b] is not None:
            out_d[b].wait()


def kernel(input, shuffle_indices):
    out_flat = _shuffle(input.reshape(-1), shuffle_indices)
    return out_flat.reshape(N_ROWS, N_COLS)


# speedup vs baseline: 3.2298x; 1.0028x over previous
import functools

import jax
import jax.numpy as jnp
from jax import lax
from jax.experimental import pallas as pl
from jax.experimental.pallas import tpu as pltpu
from jax.experimental.pallas import tpu_sc as plsc

N_ROWS = 131072
N_COLS = 128

_info = plsc.get_sparse_core_info()
NC, NS, L = _info.num_cores, _info.num_subcores, _info.num_lanes
NW = NC * NS
ROWS_PER_W = N_ROWS // NW
CHUNK_ROWS = 128
N_CHUNKS = ROWS_PER_W // CHUNK_ROWS
CHUNK_ELEMS = CHUNK_ROWS * N_COLS
G = N_COLS // L
NBUF = 2

_mesh = plsc.VectorSubcoreMesh(core_axis_name="c", subcore_axis_name="s")


@functools.partial(
    pl.kernel,
    mesh=_mesh,
    out_type=jax.ShapeDtypeStruct((N_ROWS * N_COLS,), jnp.float32),
    scratch_types=[
        pltpu.VMEM((N_COLS,), jnp.int32),
        [pltpu.VMEM((CHUNK_ELEMS,), jnp.float32) for _ in range(NBUF)],
        [pltpu.VMEM((CHUNK_ELEMS,), jnp.float32) for _ in range(NBUF)],
        [pltpu.SemaphoreType.DMA for _ in range(NBUF)],
        [pltpu.SemaphoreType.DMA for _ in range(NBUF)],
    ],
    compiler_params=pltpu.CompilerParams(needs_layout_passes=False),
)
def _shuffle(x_hbm, idx_hbm, out_hbm, idx_v, in_v, out_v, in_sem, out_sem):
    wid = lax.axis_index("s") * NC + lax.axis_index("c")
    pltpu.sync_copy(idx_hbm, idx_v)
    col_idx = [idx_v[pl.ds(g * L, L)] for g in range(G)]
    base_w = wid * (ROWS_PER_W * N_COLS)

    def in_copy(ci, b):
        return pltpu.async_copy(
            x_hbm.at[pl.ds(base_w + ci * CHUNK_ELEMS, CHUNK_ELEMS)],
            in_v[b], in_sem[b])

    def permute_chunk(src, dst):
        @plsc.parallel_loop(0, CHUNK_ROWS, step=1, unroll=8)
        def row_body(r):
            rb = r * N_COLS
            for g in range(G):
                v = plsc.load_gather(src, [col_idx[g] + rb])
                dst[pl.ds(rb + g * L, L)] = v

    in_d = [None] * NBUF
    out_d = [None] * NBUF
    in_d[0] = in_copy(0, 0)
    for ci in range(N_CHUNKS):
        b = ci % NBUF
        in_d[b].wait()
        if ci + 1 < N_CHUNKS:
            nb = (ci + 1) % NBUF
            in_d[nb] = in_copy(ci + 1, nb)
        if out_d[b] is not None:
            out_d[b].wait()
        permute_chunk(in_v[b], out_v[b])
        out_d[b] = pltpu.async_copy(
            out_v[b],
            out_hbm.at[pl.ds(base_w + ci * CHUNK_ELEMS, CHUNK_ELEMS)],
            out_sem[b])
    for b in range(NBUF):
        if out_d[b] is not None:
            out_d[b].wait()


def kernel(input, shuffle_indices):
    out_flat = _shuffle(input.reshape(-1), shuffle_indices)
    return out_flat.reshape(N_ROWS, N_COLS)
